# baseline (device time: 178490 ns/iter reference)
import functools

import jax
import jax.numpy as jnp
from jax import lax
from jax.experimental import pallas as pl
from jax.experimental.pallas import tpu as pltpu

N_DEV = 4
M = 1536
N = 1536
CHUNK = M // N_DEV


def _gelu(z):
    return 0.5 * z * (1.0 + jnp.tanh(0.7978845608 * (z + 0.044715 * z * z * z)))


def kernel(A, B):
    m, k_per = A.shape
    _, n = B.shape

    def body(a_ref, b_ref, out_ref, acc_ref, rs_buf,
             rs_send_sems, rs_recv_sems, ag_send_sems, ag_recv_sems):
        my_pos = lax.axis_index("i")
        left = (my_pos + N_DEV - 1) % N_DEV
        right = (my_pos + 1) % N_DEV

        barrier_sem = pltpu.get_barrier_semaphore()
        for nbr in [left, right]:
            pl.semaphore_signal(
                barrier_sem, inc=1,
                device_id=(nbr,), device_id_type=pl.DeviceIdType.MESH,
            )
        pl.semaphore_wait(barrier_sem, 2)

        acc_ref[...] = jnp.dot(
            a_ref[...], b_ref[...], preferred_element_type=jnp.float32
        )

        for s in range(N_DEV - 1):
            c_send = (my_pos + (N_DEV - s)) % N_DEV
            rdma = pltpu.make_async_remote_copy(
                src_ref=acc_ref.at[pl.ds(c_send * CHUNK, CHUNK), :],
                dst_ref=rs_buf.at[s],
                send_sem=rs_send_sems.at[s],
                recv_sem=rs_recv_sems.at[s],
                device_id=(right,),
                device_id_type=pl.DeviceIdType.MESH,
            )
            rdma.start()
            rdma.wait()
            c_recv = (my_pos + (N_DEV - s - 1)) % N_DEV
            acc_ref[pl.ds(c_recv * CHUNK, CHUNK), :] = (
                acc_ref[pl.ds(c_recv * CHUNK, CHUNK), :] + rs_buf[s]
            )

        own = (my_pos + 1) % N_DEV
        out_ref[pl.ds(own * CHUNK, CHUNK), :] = _gelu(
            acc_ref[pl.ds(own * CHUNK, CHUNK), :]
        )

        for s in range(N_DEV - 1):
            c_send = (my_pos + (N_DEV + 1 - s)) % N_DEV
            rdma = pltpu.make_async_remote_copy(
                src_ref=out_ref.at[pl.ds(c_send * CHUNK, CHUNK), :],
                dst_ref=out_ref.at[pl.ds(c_send * CHUNK, CHUNK), :],
                send_sem=ag_send_sems.at[s],
                recv_sem=ag_recv_sems.at[s],
                device_id=(right,),
                device_id_type=pl.DeviceIdType.MESH,
            )
            rdma.start()
            rdma.wait()

    return pl.pallas_call(
        body,
        out_shape=jax.ShapeDtypeStruct((M, N), jnp.float32),
        in_specs=[
            pl.BlockSpec(memory_space=pltpu.VMEM),
            pl.BlockSpec(memory_space=pltpu.VMEM),
        ],
        out_specs=pl.BlockSpec(memory_space=pltpu.VMEM),
        scratch_shapes=[
            pltpu.VMEM((M, N), jnp.float32),
            pltpu.VMEM((N_DEV - 1, CHUNK, N), jnp.float32),
            pltpu.SemaphoreType.DMA((N_DEV - 1,)),
            pltpu.SemaphoreType.DMA((N_DEV - 1,)),
            pltpu.SemaphoreType.DMA((N_DEV - 1,)),
            pltpu.SemaphoreType.DMA((N_DEV - 1,)),
        ],
        compiler_params=pltpu.CompilerParams(collective_id=0),
    )(A, B)


# device time: 62774 ns/iter; 2.8434x vs baseline; 2.8434x over previous
import jax
import jax.numpy as jnp
from jax import lax
from jax.experimental import pallas as pl
from jax.experimental.pallas import tpu as pltpu

N_DEV = 4
M = 1536
N = 1536
HALF = M // 2
Q = M // 4


def _gelu(z):
    return 0.5 * z * (1.0 + jnp.tanh(0.7978845608 * (z + 0.044715 * z * z * z)))


def kernel(A, B):
    def body(a_ref, b_ref, out_ref, acc_ref,
             sA1, rA1, sB1, rB1, sA2, rA2, sB2, rB2,
             gA, gB, oA, oB, send_sems, recv_sems):
        i = lax.axis_index("i")
        p1 = jnp.bitwise_xor(i, 1)
        p2 = 3 - i

        barrier_sem = pltpu.get_barrier_semaphore()
        for nbr in [p1, p2]:
            pl.semaphore_signal(
                barrier_sem, inc=1,
                device_id=(nbr,), device_id_type=pl.DeviceIdType.MESH,
            )
        pl.semaphore_wait(barrier_sem, 2)

        acc_ref[...] = jnp.dot(
            a_ref[...].astype(jnp.bfloat16),
            b_ref[...].astype(jnp.bfloat16),
            preferred_element_type=jnp.float32,
        )

        keptA = jnp.where((i == 0) | (i == 3), 0, HALF)
        sendA = HALF - keptA
        qlA = jnp.where(i >= 2, 1, 0)
        ownA = keptA + qlA * Q
        othA = keptA + (1 - qlA) * Q

        keptB = jnp.where(i <= 1, 0, HALF)
        sendB = HALF - keptB
        qlB = jnp.where(i % 2 == 1, 1, 0)
        ownB = keptB + qlB * Q
        othB = keptB + (1 - qlB) * Q

        def xfer(src, dst, sem_idx, dev):
            return pltpu.make_async_remote_copy(
                src_ref=src, dst_ref=dst,
                send_sem=send_sems.at[sem_idx], recv_sem=recv_sems.at[sem_idx],
                device_id=(dev,), device_id_type=pl.DeviceIdType.MESH,
            )

        sA1[...] = acc_ref[pl.ds(sendA, HALF), pl.ds(0, HALF)].astype(jnp.bfloat16)
        sB1[...] = acc_ref[pl.ds(sendB, HALF), pl.ds(HALF, HALF)].astype(jnp.bfloat16)
        a1 = xfer(sA1, rA1, 0, p1)
        b1 = xfer(sB1, rB1, 1, p2)
        a1.start()
        b1.start()
        a1.wait()
        b1.wait()
        acc_ref[pl.ds(keptA, HALF), pl.ds(0, HALF)] = (
            acc_ref[pl.ds(keptA, HALF), pl.ds(0, HALF)] + rA1[...].astype(jnp.float32)
        )
        acc_ref[pl.ds(keptB, HALF), pl.ds(HALF, HALF)] = (
            acc_ref[pl.ds(keptB, HALF), pl.ds(HALF, HALF)] + rB1[...].astype(jnp.float32)
        )

        sA2[...] = acc_ref[pl.ds(othA, Q), pl.ds(0, HALF)].astype(jnp.bfloat16)
        sB2[...] = acc_ref[pl.ds(othB, Q), pl.ds(HALF, HALF)].astype(jnp.bfloat16)
        a2 = xfer(sA2, rA2, 2, p2)
        b2 = xfer(sB2, rB2, 3, p1)
        a2.start()
        b2.start()
        a2.wait()
        b2.wait()

        zA = acc_ref[pl.ds(ownA, Q), pl.ds(0, HALF)] + rA2[...].astype(jnp.float32)
        gzA = _gelu(zA)
        out_ref[pl.ds(ownA, Q), pl.ds(0, HALF)] = gzA
        gA[pl.ds(qlA * Q, Q), :] = gzA.astype(jnp.bfloat16)

        zB = acc_ref[pl.ds(ownB, Q), pl.ds(HALF, HALF)] + rB2[...].astype(jnp.float32)
        gzB = _gelu(zB)
        out_ref[pl.ds(ownB, Q), pl.ds(HALF, HALF)] = gzB
        gB[pl.ds(qlB * Q, Q), :] = gzB.astype(jnp.bfloat16)

        a3 = xfer(gA.at[pl.ds(qlA * Q, Q), :], gA.at[pl.ds(qlA * Q, Q), :], 4, p2)
        b3 = xfer(gB.at[pl.ds(qlB * Q, Q), :], gB.at[pl.ds(qlB * Q, Q), :], 5, p1)
        a3.start()
        b3.start()
        a3.wait()
        b3.wait()
        out_ref[pl.ds(othA, Q), pl.ds(0, HALF)] = (
            gA[pl.ds((1 - qlA) * Q, Q), :].astype(jnp.float32)
        )
        out_ref[pl.ds(othB, Q), pl.ds(HALF, HALF)] = (
            gB[pl.ds((1 - qlB) * Q, Q), :].astype(jnp.float32)
        )

        a4 = xfer(gA, oA, 6, p1)
        b4 = xfer(gB, oB, 7, p2)
        a4.start()
        b4.start()
        a4.wait()
        b4.wait()
        out_ref[pl.ds(sendA, HALF), pl.ds(0, HALF)] = oA[...].astype(jnp.float32)
        out_ref[pl.ds(sendB, HALF), pl.ds(HALF, HALF)] = oB[...].astype(jnp.float32)

    bf16 = jnp.bfloat16
    return pl.pallas_call(
        body,
        out_shape=jax.ShapeDtypeStruct((M, N), jnp.float32),
        in_specs=[
            pl.BlockSpec(memory_space=pltpu.VMEM),
            pl.BlockSpec(memory_space=pltpu.VMEM),
        ],
        out_specs=pl.BlockSpec(memory_space=pltpu.VMEM),
        scratch_shapes=[
            pltpu.VMEM((M, N), jnp.float32),
            pltpu.VMEM((HALF, HALF), bf16),
            pltpu.VMEM((HALF, HALF), bf16),
            pltpu.VMEM((HALF, HALF), bf16),
            pltpu.VMEM((HALF, HALF), bf16),
            pltpu.VMEM((Q, HALF), bf16),
            pltpu.VMEM((Q, HALF), bf16),
            pltpu.VMEM((Q, HALF), bf16),
            pltpu.VMEM((Q, HALF), bf16),
            pltpu.VMEM((HALF, HALF), bf16),
            pltpu.VMEM((HALF, HALF), bf16),
            pltpu.VMEM((HALF, HALF), bf16),
            pltpu.VMEM((HALF, HALF), bf16),
            pltpu.SemaphoreType.DMA((8,)),
            pltpu.SemaphoreType.DMA((8,)),
        ],
        compiler_params=pltpu.CompilerParams(collective_id=0),
    )(A, B)


# device time: 58240 ns/iter; 3.0647x vs baseline; 1.0779x over previous
import jax
import jax.numpy as jnp
from jax import lax
from jax.experimental import pallas as pl
from jax.experimental.pallas import tpu as pltpu

N_DEV = 4
M = 1536
N = 1536
HALF = M // 2
Q = M // 4


def _gelu(z):
    return 0.5 * z * (1.0 + jnp.tanh(0.7978845608 * (z + 0.044715 * z * z * z)))


def kernel(A, B):
    def body(a_ref, b_ref, out_ref, acc_ref,
             sA1, rA1, sB1, rB1, sA2, rA2, sB2, rB2,
             gA, gB, oA, oB, send_sems, recv_sems):
        i = lax.axis_index("i")
        p1 = jnp.bitwise_xor(i, 1)
        p2 = 3 - i

        barrier_sem = pltpu.get_barrier_semaphore()
        for nbr in [p1, p2]:
            pl.semaphore_signal(
                barrier_sem, inc=1,
                device_id=(nbr,), device_id_type=pl.DeviceIdType.MESH,
            )
        pl.semaphore_wait(barrier_sem, 2)

        keptA = jnp.where((i == 0) | (i == 3), 0, HALF)
        sendA = HALF - keptA
        qlA = jnp.where(i >= 2, 1, 0)
        ownA = keptA + qlA * Q
        othA = keptA + (1 - qlA) * Q

        keptB = jnp.where(i <= 1, 0, HALF)
        sendB = HALF - keptB
        qlB = jnp.where(i % 2 == 1, 1, 0)
        ownB = keptB + qlB * Q
        othB = keptB + (1 - qlB) * Q

        def xfer(src, dst, sem_idx, dev):
            return pltpu.make_async_remote_copy(
                src_ref=src, dst_ref=dst,
                send_sem=send_sems.at[sem_idx], recv_sem=recv_sems.at[sem_idx],
                device_id=(dev,), device_id_type=pl.DeviceIdType.MESH,
            )

        def block(row_start, col_start):
            return jnp.dot(
                a_ref[pl.ds(row_start, HALF), :].astype(jnp.bfloat16),
                b_ref[:, pl.ds(col_start, HALF)].astype(jnp.bfloat16),
                preferred_element_type=jnp.float32,
            )

        sA1[...] = block(sendA, 0).astype(jnp.bfloat16)
        sB1[...] = block(sendB, HALF).astype(jnp.bfloat16)

        a1 = xfer(sA1, rA1, 0, p1)
        b1 = xfer(sB1, rB1, 1, p2)
        a1.start()
        b1.start()

        acc_ref[pl.ds(keptA, HALF), pl.ds(0, HALF)] = block(keptA, 0)
        acc_ref[pl.ds(keptB, HALF), pl.ds(HALF, HALF)] = block(keptB, HALF)

        a1.wait()
        b1.wait()

        sA2[...] = (
            acc_ref[pl.ds(othA, Q), pl.ds(0, HALF)]
            + rA1[pl.ds((1 - qlA) * Q, Q), :].astype(jnp.float32)
        ).astype(jnp.bfloat16)
        sB2[...] = (
            acc_ref[pl.ds(othB, Q), pl.ds(HALF, HALF)]
            + rB1[pl.ds((1 - qlB) * Q, Q), :].astype(jnp.float32)
        ).astype(jnp.bfloat16)
        a2 = xfer(sA2, rA2, 2, p2)
        b2 = xfer(sB2, rB2, 3, p1)
        a2.start()
        b2.start()

        acc_ref[pl.ds(ownA, Q), pl.ds(0, HALF)] = (
            acc_ref[pl.ds(ownA, Q), pl.ds(0, HALF)]
            + rA1[pl.ds(qlA * Q, Q), :].astype(jnp.float32)
        )
        acc_ref[pl.ds(ownB, Q), pl.ds(HALF, HALF)] = (
            acc_ref[pl.ds(ownB, Q), pl.ds(HALF, HALF)]
            + rB1[pl.ds(qlB * Q, Q), :].astype(jnp.float32)
        )

        a2.wait()
        b2.wait()

        gzA = _gelu(
            acc_ref[pl.ds(ownA, Q), pl.ds(0, HALF)] + rA2[...].astype(jnp.float32)
        )
        out_ref[pl.ds(ownA, Q), pl.ds(0, HALF)] = gzA
        gA[pl.ds(qlA * Q, Q), :] = gzA.astype(jnp.bfloat16)

        gzB = _gelu(
            acc_ref[pl.ds(ownB, Q), pl.ds(HALF, HALF)] + rB2[...].astype(jnp.float32)
        )
        out_ref[pl.ds(ownB, Q), pl.ds(HALF, HALF)] = gzB
        gB[pl.ds(qlB * Q, Q), :] = gzB.astype(jnp.bfloat16)

        a3 = xfer(gA.at[pl.ds(qlA * Q, Q), :], gA.at[pl.ds(qlA * Q, Q), :], 4, p2)
        b3 = xfer(gB.at[pl.ds(qlB * Q, Q), :], gB.at[pl.ds(qlB * Q, Q), :], 5, p1)
        a4a = xfer(gA.at[pl.ds(qlA * Q, Q), :], oA.at[pl.ds(qlA * Q, Q), :], 6, p1)
        b4a = xfer(gB.at[pl.ds(qlB * Q, Q), :], oB.at[pl.ds(qlB * Q, Q), :], 7, p2)
        a3.start()
        b3.start()
        a4a.start()
        b4a.start()

        a3.wait()
        b3.wait()

        a4b = xfer(
            gA.at[pl.ds((1 - qlA) * Q, Q), :], oA.at[pl.ds((1 - qlA) * Q, Q), :],
            8, p1,
        )
        b4b = xfer(
            gB.at[pl.ds((1 - qlB) * Q, Q), :], oB.at[pl.ds((1 - qlB) * Q, Q), :],
            9, p2,
        )
        a4b.start()
        b4b.start()

        out_ref[pl.ds(othA, Q), pl.ds(0, HALF)] = (
            gA[pl.ds((1 - qlA) * Q, Q), :].astype(jnp.float32)
        )
        out_ref[pl.ds(othB, Q), pl.ds(HALF, HALF)] = (
            gB[pl.ds((1 - qlB) * Q, Q), :].astype(jnp.float32)
        )

        a4a.wait()
        b4a.wait()
        out_ref[pl.ds(sendA + qlA * Q, Q), pl.ds(0, HALF)] = (
            oA[pl.ds(qlA * Q, Q), :].astype(jnp.float32)
        )
        out_ref[pl.ds(sendB + (1 - qlB) * Q, Q), pl.ds(HALF, HALF)] = (
            oB[pl.ds((1 - qlB) * Q, Q), :].astype(jnp.float32)
        )

        a4b.wait()
        b4b.wait()
        out_ref[pl.ds(sendA + (1 - qlA) * Q, Q), pl.ds(0, HALF)] = (
            oA[pl.ds((1 - qlA) * Q, Q), :].astype(jnp.float32)
        )
        out_ref[pl.ds(sendB + qlB * Q, Q), pl.ds(HALF, HALF)] = (
            oB[pl.ds(qlB * Q, Q), :].astype(jnp.float32)
        )

    bf16 = jnp.bfloat16
    return pl.pallas_call(
        body,
        out_shape=jax.ShapeDtypeStruct((M, N), jnp.float32),
        in_specs=[
            pl.BlockSpec(memory_space=pltpu.VMEM),
            pl.BlockSpec(memory_space=pltpu.VMEM),
        ],
        out_specs=pl.BlockSpec(memory_space=pltpu.VMEM),
        scratch_shapes=[
            pltpu.VMEM((M, N), jnp.float32),
            pltpu.VMEM((HALF, HALF), bf16),
            pltpu.VMEM((HALF, HALF), bf16),
            pltpu.VMEM((HALF, HALF), bf16),
            pltpu.VMEM((HALF, HALF), bf16),
            pltpu.VMEM((Q, HALF), bf16),
            pltpu.VMEM((Q, HALF), bf16),
            pltpu.VMEM((Q, HALF), bf16),
            pltpu.VMEM((Q, HALF), bf16),
            pltpu.VMEM((HALF, HALF), bf16),
            pltpu.VMEM((HALF, HALF), bf16),
            pltpu.VMEM((HALF, HALF), bf16),
            pltpu.VMEM((HALF, HALF), bf16),
            pltpu.SemaphoreType.DMA((10,)),
            pltpu.SemaphoreType.DMA((10,)),
        ],
        compiler_params=pltpu.CompilerParams(collective_id=0),
    )(A, B)


# device time: 55471 ns/iter; 3.2177x vs baseline; 1.0499x over previous
import jax
import jax.numpy as jnp
from jax import lax
from jax.experimental import pallas as pl
from jax.experimental.pallas import tpu as pltpu

N_DEV = 4
M = 1536
N = 1536
HALF = M // 2
Q = M // 4


def _gelu(z):
    return 0.5 * z * (1.0 + jnp.tanh(0.7978845608 * (z + 0.044715 * z * z * z)))


def kernel(A, B):
    def body(a_ref, b_ref, out_ref, acc_ref,
             sA1, rA1, sB1, rB1, sA2, rA2, sB2, rB2,
             gA, gB, oA, oB, send_sems, recv_sems):
        i = lax.axis_index("i")
        p1 = jnp.bitwise_xor(i, 1)
        p2 = 3 - i

        barrier_sem = pltpu.get_barrier_semaphore()
        for nbr in [p1, p2]:
            pl.semaphore_signal(
                barrier_sem, inc=1,
                device_id=(nbr,), device_id_type=pl.DeviceIdType.MESH,
            )
        pl.semaphore_wait(barrier_sem, 2)

        keptA = jnp.where((i == 0) | (i == 3), 0, HALF)
        sendA = HALF - keptA
        qlA = jnp.where(i >= 2, 1, 0)
        ownA = keptA + qlA * Q
        othA = keptA + (1 - qlA) * Q

        keptB = jnp.where(i <= 1, 0, HALF)
        sendB = HALF - keptB
        qlB = jnp.where(i % 2 == 1, 1, 0)
        ownB = keptB + qlB * Q
        othB = keptB + (1 - qlB) * Q

        def xfer(src, dst, sem_idx, dev):
            return pltpu.make_async_remote_copy(
                src_ref=src, dst_ref=dst,
                send_sem=send_sems.at[sem_idx], recv_sem=recv_sems.at[sem_idx],
                device_id=(dev,), device_id_type=pl.DeviceIdType.MESH,
            )

        def qblock(row_start, col_start):
            return jnp.dot(
                a_ref[pl.ds(row_start, Q), :].astype(jnp.bfloat16),
                b_ref[:, pl.ds(col_start, HALF)].astype(jnp.bfloat16),
                preferred_element_type=jnp.float32,
            )

        sA1[pl.ds((1 - qlA) * Q, Q), :] = (
            qblock(sendA + (1 - qlA) * Q, 0).astype(jnp.bfloat16)
        )
        a1f = xfer(sA1.at[pl.ds((1 - qlA) * Q, Q), :],
                   rA1.at[pl.ds((1 - qlA) * Q, Q), :], 0, p1)
        a1f.start()

        sB1[pl.ds(qlB * Q, Q), :] = (
            qblock(sendB + qlB * Q, HALF).astype(jnp.bfloat16)
        )
        b1f = xfer(sB1.at[pl.ds(qlB * Q, Q), :],
                   rB1.at[pl.ds(qlB * Q, Q), :], 2, p2)
        b1f.start()

        sA1[pl.ds(qlA * Q, Q), :] = (
            qblock(sendA + qlA * Q, 0).astype(jnp.bfloat16)
        )
        a1s = xfer(sA1.at[pl.ds(qlA * Q, Q), :],
                   rA1.at[pl.ds(qlA * Q, Q), :], 1, p1)
        a1s.start()

        sB1[pl.ds((1 - qlB) * Q, Q), :] = (
            qblock(sendB + (1 - qlB) * Q, HALF).astype(jnp.bfloat16)
        )
        b1s = xfer(sB1.at[pl.ds((1 - qlB) * Q, Q), :],
                   rB1.at[pl.ds((1 - qlB) * Q, Q), :], 3, p2)
        b1s.start()

        acc_ref[pl.ds(othA, Q), pl.ds(0, HALF)] = qblock(othA, 0)
        acc_ref[pl.ds(othB, Q), pl.ds(HALF, HALF)] = qblock(othB, HALF)

        a1f.wait()
        sA2[...] = (
            acc_ref[pl.ds(othA, Q), pl.ds(0, HALF)]
            + rA1[pl.ds((1 - qlA) * Q, Q), :].astype(jnp.float32)
        ).astype(jnp.bfloat16)
        a2 = xfer(sA2, rA2, 4, p2)
        a2.start()

        b1f.wait()
        sB2[...] = (
            acc_ref[pl.ds(othB, Q), pl.ds(HALF, HALF)]
            + rB1[pl.ds((1 - qlB) * Q, Q), :].astype(jnp.float32)
        ).astype(jnp.bfloat16)
        b2 = xfer(sB2, rB2, 5, p1)
        b2.start()

        acc_ref[pl.ds(ownA, Q), pl.ds(0, HALF)] = qblock(ownA, 0)
        acc_ref[pl.ds(ownB, Q), pl.ds(HALF, HALF)] = qblock(ownB, HALF)

        a1s.wait()
        acc_ref[pl.ds(ownA, Q), pl.ds(0, HALF)] = (
            acc_ref[pl.ds(ownA, Q), pl.ds(0, HALF)]
            + rA1[pl.ds(qlA * Q, Q), :].astype(jnp.float32)
        )
        b1s.wait()
        acc_ref[pl.ds(ownB, Q), pl.ds(HALF, HALF)] = (
            acc_ref[pl.ds(ownB, Q), pl.ds(HALF, HALF)]
            + rB1[pl.ds(qlB * Q, Q), :].astype(jnp.float32)
        )

        a2.wait()
        b2.wait()

        gzA = _gelu(
            acc_ref[pl.ds(ownA, Q), pl.ds(0, HALF)] + rA2[...].astype(jnp.float32)
        )
        out_ref[pl.ds(ownA, Q), pl.ds(0, HALF)] = gzA
        gA[pl.ds(qlA * Q, Q), :] = gzA.astype(jnp.bfloat16)

        gzB = _gelu(
            acc_ref[pl.ds(ownB, Q), pl.ds(HALF, HALF)] + rB2[...].astype(jnp.float32)
        )
        out_ref[pl.ds(ownB, Q), pl.ds(HALF, HALF)] = gzB
        gB[pl.ds(qlB * Q, Q), :] = gzB.astype(jnp.bfloat16)

        a3 = xfer(gA.at[pl.ds(qlA * Q, Q), :], gA.at[pl.ds(qlA * Q, Q), :], 6, p2)
        b3 = xfer(gB.at[pl.ds(qlB * Q, Q), :], gB.at[pl.ds(qlB * Q, Q), :], 7, p1)
        a4a = xfer(gA.at[pl.ds(qlA * Q, Q), :], oA.at[pl.ds(qlA * Q, Q), :], 8, p1)
        b4a = xfer(gB.at[pl.ds(qlB * Q, Q), :], oB.at[pl.ds(qlB * Q, Q), :], 9, p2)
        a3.start()
        b3.start()
        a4a.start()
        b4a.start()

        a3.wait()
        b3.wait()

        a4b = xfer(
            gA.at[pl.ds((1 - qlA) * Q, Q), :], oA.at[pl.ds((1 - qlA) * Q, Q), :],
            10, p1,
        )
        b4b = xfer(
            gB.at[pl.ds((1 - qlB) * Q, Q), :], oB.at[pl.ds((1 - qlB) * Q, Q), :],
            11, p2,
        )
        a4b.start()
        b4b.start()

        out_ref[pl.ds(othA, Q), pl.ds(0, HALF)] = (
            gA[pl.ds((1 - qlA) * Q, Q), :].astype(jnp.float32)
        )
        out_ref[pl.ds(othB, Q), pl.ds(HALF, HALF)] = (
            gB[pl.ds((1 - qlB) * Q, Q), :].astype(jnp.float32)
        )

        a4a.wait()
        b4a.wait()
        out_ref[pl.ds(sendA + qlA * Q, Q), pl.ds(0, HALF)] = (
            oA[pl.ds(qlA * Q, Q), :].astype(jnp.float32)
        )
        out_ref[pl.ds(sendB + (1 - qlB) * Q, Q), pl.ds(HALF, HALF)] = (
            oB[pl.ds((1 - qlB) * Q, Q), :].astype(jnp.float32)
        )

        a4b.wait()
        b4b.wait()
        out_ref[pl.ds(sendA + (1 - qlA) * Q, Q), pl.ds(0, HALF)] = (
            oA[pl.ds((1 - qlA) * Q, Q), :].astype(jnp.float32)
        )
        out_ref[pl.ds(sendB + qlB * Q, Q), pl.ds(HALF, HALF)] = (
            oB[pl.ds(qlB * Q, Q), :].astype(jnp.float32)
        )

    bf16 = jnp.bfloat16
    return pl.pallas_call(
        body,
        out_shape=jax.ShapeDtypeStruct((M, N), jnp.float32),
        in_specs=[
            pl.BlockSpec(memory_space=pltpu.VMEM),
            pl.BlockSpec(memory_space=pltpu.VMEM),
        ],
        out_specs=pl.BlockSpec(memory_space=pltpu.VMEM),
        scratch_shapes=[
            pltpu.VMEM((M, N), jnp.float32),
            pltpu.VMEM((HALF, HALF), bf16),
            pltpu.VMEM((HALF, HALF), bf16),
            pltpu.VMEM((HALF, HALF), bf16),
            pltpu.VMEM((HALF, HALF), bf16),
            pltpu.VMEM((Q, HALF), bf16),
            pltpu.VMEM((Q, HALF), bf16),
            pltpu.VMEM((Q, HALF), bf16),
            pltpu.VMEM((Q, HALF), bf16),
            pltpu.VMEM((HALF, HALF), bf16),
            pltpu.VMEM((HALF, HALF), bf16),
            pltpu.VMEM((HALF, HALF), bf16),
            pltpu.VMEM((HALF, HALF), bf16),
            pltpu.SemaphoreType.DMA((12,)),
            pltpu.SemaphoreType.DMA((12,)),
        ],
        compiler_params=pltpu.CompilerParams(collective_id=0),
    )(A, B)


# device time: 53231 ns/iter; 3.3531x vs baseline; 1.0421x over previous
import jax
import jax.numpy as jnp
from jax import lax
from jax.experimental import pallas as pl
from jax.experimental.pallas import tpu as pltpu

N_DEV = 4
M = 1536
N = 1536
HALF = M // 2
Q = M // 4


def _gelu(z):
    return 0.5 * z * (1.0 + jnp.tanh(0.7978845608 * (z + 0.044715 * z * z * z)))


def kernel(A, B):
    def body(a_ref, b_ref, out_ref, acc_ref,
             sA1, rA1, sB1, rB1, sA2, rA2, sB2, rB2,
             gA, gB, oA, oB, send_sems, recv_sems):
        i = lax.axis_index("i")
        p1 = jnp.bitwise_xor(i, 1)
        p2 = 3 - i

        barrier_sem = pltpu.get_barrier_semaphore()
        for nbr in [p1, p2]:
            pl.semaphore_signal(
                barrier_sem, inc=1,
                device_id=(nbr,), device_id_type=pl.DeviceIdType.MESH,
            )
        pl.semaphore_wait(barrier_sem, 2)

        keptA = jnp.where((i == 0) | (i == 3), 0, HALF)
        sendA = HALF - keptA
        qlA = jnp.where(i >= 2, 1, 0)
        ownA = keptA + qlA * Q
        othA = keptA + (1 - qlA) * Q

        keptB = jnp.where(i <= 1, 0, HALF)
        sendB = HALF - keptB
        qlB = jnp.where(i % 2 == 1, 1, 0)
        ownB = keptB + qlB * Q
        othB = keptB + (1 - qlB) * Q

        bf16 = jnp.bfloat16
        f32 = jnp.float32

        def xfer(src, dst, sem_idx, dev):
            return pltpu.make_async_remote_copy(
                src_ref=src, dst_ref=dst,
                send_sem=send_sems.at[sem_idx], recv_sem=recv_sems.at[sem_idx],
                device_id=(dev,), device_id_type=pl.DeviceIdType.MESH,
            )

        def hblock(row_start, col_start):
            return jnp.dot(
                a_ref[pl.ds(row_start, HALF), :].astype(bf16),
                b_ref[:, pl.ds(col_start, Q)].astype(bf16),
                preferred_element_type=f32,
            )

        def mk(bfly, s):
            if bfly == "A":
                col_g = s * Q
                kept, send, ql, own, oth = keptA, sendA, qlA, ownA, othA
                s1, r1, s2, r2, g, o = sA1, rA1, sA2, rA2, gA, oA
                d1, d2 = p1, p2
                ob4a, ob4b = ql, 1 - ql
            else:
                col_g = HALF + s * Q
                kept, send, ql, own, oth = keptB, sendB, qlB, ownB, othB
                s1, r1, s2, r2, g, o = sB1, rB1, sB2, rB2, gB, oB
                d1, d2 = p2, p1
                ob4a, ob4b = 1 - ql, ql
            col_b = s * Q
            base = {"A": 0, "B": 1}[bfly] + 2 * s
            return dict(
                col_g=col_g, col_b=col_b, kept=kept, send=send, ql=ql,
                own=own, oth=oth, s1=s1, r1=r1, s2=s2, r2=r2, g=g, o=o,
                d1=d1, d2=d2, ob4a=ob4a, ob4b=ob4b, base=base,
            )

        ctxs = {(b, s): mk(b, s) for b in ("A", "B") for s in (0, 1)}

        def sem(c, phase):
            return c["base"] + 4 * phase

        def compute_send(c):
            c["s1"][:, pl.ds(c["col_b"], Q)] = (
                hblock(c["send"], c["col_g"]).astype(bf16)
            )

        def start_p1(c):
            r = xfer(c["s1"].at[:, pl.ds(c["col_b"], Q)],
                     c["r1"].at[:, pl.ds(c["col_b"], Q)], sem(c, 0), c["d1"])
            r.start()
            return r

        def compute_kept(c):
            acc_ref[pl.ds(c["kept"], HALF), pl.ds(c["col_g"], Q)] = (
                hblock(c["kept"], c["col_g"])
            )

        def stage_start_p2(c):
            c["s2"][:, pl.ds(c["col_b"], Q)] = (
                acc_ref[pl.ds(c["oth"], Q), pl.ds(c["col_g"], Q)]
                + c["r1"][pl.ds((1 - c["ql"]) * Q, Q),
                          pl.ds(c["col_b"], Q)].astype(f32)
            ).astype(bf16)
            r = xfer(c["s2"].at[:, pl.ds(c["col_b"], Q)],
                     c["r2"].at[:, pl.ds(c["col_b"], Q)], sem(c, 1), c["d2"])
            r.start()
            return r

        def gelu_start_p3_p4a(c):
            gz = _gelu(
                acc_ref[pl.ds(c["own"], Q), pl.ds(c["col_g"], Q)]
                + c["r1"][pl.ds(c["ql"] * Q, Q),
                          pl.ds(c["col_b"], Q)].astype(f32)
                + c["r2"][:, pl.ds(c["col_b"], Q)].astype(f32)
            )
            out_ref[pl.ds(c["own"], Q), pl.ds(c["col_g"], Q)] = gz
            c["g"][pl.ds(c["ql"] * Q, Q), pl.ds(c["col_b"], Q)] = gz.astype(bf16)
            r3 = xfer(c["g"].at[pl.ds(c["ql"] * Q, Q), pl.ds(c["col_b"], Q)],
                      c["g"].at[pl.ds(c["ql"] * Q, Q), pl.ds(c["col_b"], Q)],
                      sem(c, 2), c["d2"])
            r4a = xfer(c["g"].at[pl.ds(c["ql"] * Q, Q), pl.ds(c["col_b"], Q)],
                       c["o"].at[pl.ds(c["ql"] * Q, Q), pl.ds(c["col_b"], Q)],
                       sem(c, 3), c["d1"])
            r3.start()
            r4a.start()
            return r3, r4a

        def start_p4b_upcast_p3(c):
            r4b = xfer(
                c["g"].at[pl.ds((1 - c["ql"]) * Q, Q), pl.ds(c["col_b"], Q)],
                c["o"].at[pl.ds((1 - c["ql"]) * Q, Q), pl.ds(c["col_b"], Q)],
                sem(c, 4), c["d1"])
            r4b.start()
            out_ref[pl.ds(c["oth"], Q), pl.ds(c["col_g"], Q)] = (
                c["g"][pl.ds((1 - c["ql"]) * Q, Q),
                       pl.ds(c["col_b"], Q)].astype(f32)
            )
            return r4b

        def upcast_p4(c, slot):
            out_ref[pl.ds(c["send"] + slot * Q, Q), pl.ds(c["col_g"], Q)] = (
                c["o"][pl.ds(slot * Q, Q), pl.ds(c["col_b"], Q)].astype(f32)
            )

        A0, B0, A1, B1 = ctxs["A", 0], ctxs["B", 0], ctxs["A", 1], ctxs["B", 1]
        order = [A0, B0, A1, B1]

        p1s = {}
        for c in order:
            compute_send(c)
            p1s[id(c)] = start_p1(c)

        for c in order:
            compute_kept(c)

        p2s = {}
        for c in order:
            p1s[id(c)].wait()
            p2s[id(c)] = stage_start_p2(c)

        p34 = {}
        for c in order:
            p2s[id(c)].wait()
            p34[id(c)] = gelu_start_p3_p4a(c)

        p4bs = {}
        for c in order:
            p34[id(c)][0].wait()
            p4bs[id(c)] = start_p4b_upcast_p3(c)

        for c in order:
            p34[id(c)][1].wait()
            upcast_p4(c, c["ob4a"])
        for c in order:
            p4bs[id(c)].wait()
            upcast_p4(c, c["ob4b"])

    bf16 = jnp.bfloat16
    return pl.pallas_call(
        body,
        out_shape=jax.ShapeDtypeStruct((M, N), jnp.float32),
        in_specs=[
            pl.BlockSpec(memory_space=pltpu.VMEM),
            pl.BlockSpec(memory_space=pltpu.VMEM),
        ],
        out_specs=pl.BlockSpec(memory_space=pltpu.VMEM),
        scratch_shapes=[
            pltpu.VMEM((M, N), jnp.float32),
            pltpu.VMEM((HALF, HALF), bf16),
            pltpu.VMEM((HALF, HALF), bf16),
            pltpu.VMEM((HALF, HALF), bf16),
            pltpu.VMEM((HALF, HALF), bf16),
            pltpu.VMEM((Q, HALF), bf16),
            pltpu.VMEM((Q, HALF), bf16),
            pltpu.VMEM((Q, HALF), bf16),
            pltpu.VMEM((Q, HALF), bf16),
            pltpu.VMEM((HALF, HALF), bf16),
            pltpu.VMEM((HALF, HALF), bf16),
            pltpu.VMEM((HALF, HALF), bf16),
            pltpu.VMEM((HALF, HALF), bf16),
            pltpu.SemaphoreType.DMA((20,)),
            pltpu.SemaphoreType.DMA((20,)),
        ],
        compiler_params=pltpu.CompilerParams(collective_id=0),
    )(A, B)
